# baseline (speedup 1.0000x reference)
import jax
import jax.numpy as jnp
from jax import lax
from jax.experimental import pallas as pl
from jax.experimental.pallas import tpu as pltpu
from jax.experimental.pallas import tpu_sc as plsc

NUM_FACTOR = 32
NUM_PAIR = NUM_FACTOR // 2
BATCH = 16384
NC = 2
NS = 16
L = 16
NW = NC * NS
B_PER_W = BATCH // NW


def _mf_body(user_hbm, item_hbm, p_hbm, q_hbm, bu_hbm, bi_hbm, out_hbm,
             uidx_v, iidx_v, prows_v, qrows_v, bu_v, bi_v, out_v,
             sem_p, sem_q, sem_bu, sem_bi):
    wid = lax.axis_index("s") * NC + lax.axis_index("c")
    base = wid * B_PER_W

    pltpu.sync_copy(user_hbm.at[pl.ds(base, B_PER_W)], uidx_v)
    pltpu.sync_copy(item_hbm.at[pl.ds(base, B_PER_W)], iidx_v)

    cp_p = pltpu.async_copy(p_hbm.at[uidx_v], prows_v, sem_p)
    cp_q = pltpu.async_copy(q_hbm.at[iidx_v], qrows_v, sem_q)
    cp_bu = pltpu.async_copy(bu_hbm.at[uidx_v], bu_v, sem_bu)
    cp_bi = pltpu.async_copy(bi_hbm.at[iidx_v], bi_v, sem_bi)
    cp_p.wait()
    cp_q.wait()
    cp_bu.wait()
    cp_bi.wait()

    lanes = lax.iota(jnp.int32, L)

    def group(g, _):
        rows = g * L + lanes
        accs = [jnp.zeros((L,), jnp.float32) for _ in range(4)]
        for j in range(NUM_PAIR):
            cols = jnp.full((L,), j, jnp.int32)
            pv = plsc.load_gather(prows_v, [rows, cols])
            qv = plsc.load_gather(qrows_v, [rows, cols])
            pa, pb = plsc.unpack(plsc.bitcast(pv, jnp.bfloat16),
                                 format=plsc.PackFormat.INTERLEAVED,
                                 preferred_element_type=jnp.float32)
            qa, qb = plsc.unpack(plsc.bitcast(qv, jnp.bfloat16),
                                 format=plsc.PackFormat.INTERLEAVED,
                                 preferred_element_type=jnp.float32)
            accs[(2 * j) % 4] = accs[(2 * j) % 4] + pa * qa
            accs[(2 * j + 1) % 4] = accs[(2 * j + 1) % 4] + pb * qb
        dot = (accs[0] + accs[1]) + (accs[2] + accs[3])
        out_v[pl.ds(g * L, L)] = dot + bu_v[pl.ds(g * L, L)] + bi_v[pl.ds(g * L, L)]
        return 0

    lax.fori_loop(0, B_PER_W // L, group, 0)

    pltpu.sync_copy(out_v, out_hbm.at[pl.ds(base, B_PER_W)])


@jax.jit
def _mf(user, item, p, q, b_u, b_i):
    pbits = jax.lax.bitcast_convert_type(
        p.astype(jnp.bfloat16).reshape(p.shape[0], NUM_PAIR, 2), jnp.int32)
    qbits = jax.lax.bitcast_convert_type(
        q.astype(jnp.bfloat16).reshape(q.shape[0], NUM_PAIR, 2), jnp.int32)
    mesh = plsc.VectorSubcoreMesh(
        core_axis_name="c", subcore_axis_name="s",
        num_cores=NC, num_subcores=NS)
    return pl.kernel(
        _mf_body,
        out_type=jax.ShapeDtypeStruct((BATCH,), jnp.float32),
        mesh=mesh,
        compiler_params=pltpu.CompilerParams(
            needs_layout_passes=False, use_tc_tiling_on_sc=False),
        scratch_types=[
            pltpu.VMEM((B_PER_W,), jnp.int32),
            pltpu.VMEM((B_PER_W,), jnp.int32),
            pltpu.VMEM((B_PER_W, NUM_PAIR), jnp.int32),
            pltpu.VMEM((B_PER_W, NUM_PAIR), jnp.int32),
            pltpu.VMEM((B_PER_W,), jnp.float32),
            pltpu.VMEM((B_PER_W,), jnp.float32),
            pltpu.VMEM((B_PER_W,), jnp.float32),
            pltpu.SemaphoreType.DMA,
            pltpu.SemaphoreType.DMA,
            pltpu.SemaphoreType.DMA,
            pltpu.SemaphoreType.DMA,
        ],
    )(user, item, pbits, qbits, b_u, b_i)


def kernel(user, item, p, q, b_u, b_i):
    return _mf(user, item, p, q, b_u, b_i)


# speedup vs baseline: 2.1775x; 2.1775x over previous
import jax
import jax.numpy as jnp
from jax import lax
from jax.experimental import pallas as pl
from jax.experimental.pallas import tpu as pltpu
from jax.experimental.pallas import tpu_sc as plsc

NUM_FACTOR = 32
BATCH = 16384
NC = 2
NS = 16
L = 16
NW = NC * NS
B_PER_W = BATCH // NW


def _mf_body(user_hbm, item_hbm, p_hbm, q_hbm, bu_hbm, bi_hbm, out_hbm,
             uidx_v, iidx_v, prows_v, qrows_v, bu_v, bi_v, out_v,
             sem_p, sem_q, sem_bu, sem_bi):
    wid = lax.axis_index("s") * NC + lax.axis_index("c")
    base = wid * B_PER_W

    pltpu.sync_copy(user_hbm.at[pl.ds(base, B_PER_W)], uidx_v)
    pltpu.sync_copy(item_hbm.at[pl.ds(base, B_PER_W)], iidx_v)

    cp_p = pltpu.async_copy(p_hbm.at[uidx_v], prows_v, sem_p)
    cp_q = pltpu.async_copy(q_hbm.at[iidx_v], qrows_v, sem_q)
    cp_bu = pltpu.async_copy(bu_hbm.at[uidx_v], bu_v, sem_bu)
    cp_bi = pltpu.async_copy(bi_hbm.at[iidx_v], bi_v, sem_bi)
    cp_p.wait()
    cp_q.wait()
    cp_bu.wait()
    cp_bi.wait()

    lanes = lax.iota(jnp.int32, L)

    def group(g, _):
        rows = g * L + lanes
        accs = [jnp.zeros((L,), jnp.float32) for _ in range(4)]
        for f in range(NUM_FACTOR):
            cols = jnp.full((L,), f, jnp.int32)
            pv = plsc.load_gather(prows_v, [rows, cols])
            qv = plsc.load_gather(qrows_v, [rows, cols])
            accs[f % 4] = accs[f % 4] + pv * qv
        dot = (accs[0] + accs[1]) + (accs[2] + accs[3])
        out_v[pl.ds(g * L, L)] = dot + bu_v[pl.ds(g * L, L)] + bi_v[pl.ds(g * L, L)]
        return 0

    lax.fori_loop(0, B_PER_W // L, group, 0)

    pltpu.sync_copy(out_v, out_hbm.at[pl.ds(base, B_PER_W)])


@jax.jit
def _mf(user, item, p, q, b_u, b_i):
    mesh = plsc.VectorSubcoreMesh(
        core_axis_name="c", subcore_axis_name="s",
        num_cores=NC, num_subcores=NS)
    return pl.kernel(
        _mf_body,
        out_type=jax.ShapeDtypeStruct((BATCH,), jnp.float32),
        mesh=mesh,
        compiler_params=pltpu.CompilerParams(
            needs_layout_passes=False, use_tc_tiling_on_sc=False),
        scratch_types=[
            pltpu.VMEM((B_PER_W,), jnp.int32),
            pltpu.VMEM((B_PER_W,), jnp.int32),
            pltpu.VMEM((B_PER_W, NUM_FACTOR), jnp.float32),
            pltpu.VMEM((B_PER_W, NUM_FACTOR), jnp.float32),
            pltpu.VMEM((B_PER_W,), jnp.float32),
            pltpu.VMEM((B_PER_W,), jnp.float32),
            pltpu.VMEM((B_PER_W,), jnp.float32),
            pltpu.SemaphoreType.DMA,
            pltpu.SemaphoreType.DMA,
            pltpu.SemaphoreType.DMA,
            pltpu.SemaphoreType.DMA,
        ],
    )(user, item, p, q, b_u, b_i)


def kernel(user, item, p, q, b_u, b_i):
    return _mf(user, item, p, q, b_u, b_i)


# speedup vs baseline: 2.9710x; 1.3644x over previous
import jax
import jax.numpy as jnp
from jax import lax
from jax.experimental import pallas as pl
from jax.experimental.pallas import tpu as pltpu
from jax.experimental.pallas import tpu_sc as plsc

NUM_FACTOR = 32
BATCH = 16384
NUM_ROWS = 1000000
NC = 2
NS = 16
L = 16
NW = NC * NS
B_PER_W = BATCH // NW
N_GATHER = B_PER_W * NUM_FACTOR
BLK = 2048
NBLK = (NUM_ROWS + BLK - 1) // BLK


def _detile_body(x_ref, o_ref):
    o_ref[...] = x_ref[...].reshape(NUM_FACTOR, BLK // 128, 128)


def _detile(pt):
    return pl.pallas_call(
        _detile_body,
        grid=(NBLK,),
        in_specs=[pl.BlockSpec((NUM_FACTOR, BLK), lambda k: (0, k))],
        out_specs=pl.BlockSpec((NUM_FACTOR, BLK // 128, 128),
                               lambda k: (k, 0, 0)),
        out_shape=jax.ShapeDtypeStruct((NBLK * NUM_FACTOR, BLK // 128, 128),
                                       jnp.float32),
    )(pt)


def _mf_body(user_hbm, item_hbm, fp_hbm, fq_hbm, bu_hbm, bi_hbm, out_hbm,
             uidx_v, iidx_v, ids_p, ids_q, dp_v, dq_v,
             bu_v, bi_v, out_v,
             sem_bu, sem_bi, sem_p, sem_q):
    wid = lax.axis_index("s") * NC + lax.axis_index("c")
    base = wid * B_PER_W

    pltpu.sync_copy(user_hbm.at[pl.ds(base, B_PER_W)], uidx_v)
    pltpu.sync_copy(item_hbm.at[pl.ds(base, B_PER_W)], iidx_v)

    copies = [
        pltpu.async_copy(bu_hbm.at[uidx_v], bu_v, sem_bu),
        pltpu.async_copy(bi_hbm.at[iidx_v], bi_v, sem_bi),
    ]

    def build(g, _):
        for idxv, ids in ((uidx_v, ids_p), (iidx_v, ids_q)):
            u16 = idxv[pl.ds(g * L, L)]
            a0 = (u16 >> 11) * (NUM_FACTOR * BLK) + (u16 & (BLK - 1))
            for f in range(NUM_FACTOR):
                ids[pl.ds(f * B_PER_W + g * L, L)] = a0 + f * BLK
        return 0

    lax.fori_loop(0, B_PER_W // L, build, 0)

    copies.append(pltpu.async_copy(fp_hbm.at[ids_p], dp_v, sem_p))
    copies.append(pltpu.async_copy(fq_hbm.at[ids_q], dq_v, sem_q))
    for c in copies:
        c.wait()

    def dot(g, _):
        sl = pl.ds(g * L, L)
        acc = bu_v[sl] + bi_v[sl]
        for f in range(NUM_FACTOR):
            pv = dp_v[pl.ds(f * B_PER_W + g * L, L)]
            qv = dq_v[pl.ds(f * B_PER_W + g * L, L)]
            acc = acc + pv * qv
        out_v[sl] = acc
        return 0

    lax.fori_loop(0, B_PER_W // L, dot, 0)

    pltpu.sync_copy(out_v, out_hbm.at[pl.ds(base, B_PER_W)])


@jax.jit
def _mf(user, item, p, q, b_u, b_i):
    fp = _detile(jnp.transpose(p)).reshape(NBLK * NUM_FACTOR * (BLK // 128) * 128)
    fq = _detile(jnp.transpose(q)).reshape(NBLK * NUM_FACTOR * (BLK // 128) * 128)
    mesh = plsc.VectorSubcoreMesh(
        core_axis_name="c", subcore_axis_name="s",
        num_cores=NC, num_subcores=NS)
    return pl.kernel(
        _mf_body,
        out_type=jax.ShapeDtypeStruct((BATCH,), jnp.float32),
        mesh=mesh,
        compiler_params=pltpu.CompilerParams(
            needs_layout_passes=False, use_tc_tiling_on_sc=False),
        scratch_types=[
            pltpu.VMEM((B_PER_W,), jnp.int32),
            pltpu.VMEM((B_PER_W,), jnp.int32),
            pltpu.VMEM((N_GATHER,), jnp.int32),
            pltpu.VMEM((N_GATHER,), jnp.int32),
            pltpu.VMEM((N_GATHER,), jnp.float32),
            pltpu.VMEM((N_GATHER,), jnp.float32),
            pltpu.VMEM((B_PER_W,), jnp.float32),
            pltpu.VMEM((B_PER_W,), jnp.float32),
            pltpu.VMEM((B_PER_W,), jnp.float32),
        ] + [pltpu.SemaphoreType.DMA] * 4,
    )(user, item, fp, fq, b_u, b_i)


def kernel(user, item, p, q, b_u, b_i):
    return _mf(user, item, p, q, b_u, b_i)


# speedup vs baseline: 8.1623x; 2.7473x over previous
import jax
import jax.numpy as jnp
from jax import lax
from jax.experimental import pallas as pl
from jax.experimental.pallas import tpu as pltpu
from jax.experimental.pallas import tpu_sc as plsc

NUM_FACTOR = 32
BATCH = 16384
NUM_ROWS = 1000000
NC = 2
NS = 16
L = 16
NW = NC * NS
B_PER_W = BATCH // NW
N_GATHER = B_PER_W * NUM_FACTOR
BLK = 16384
NBLK = (NUM_ROWS + BLK - 1) // BLK


def _detile_body(x_ref, o_ref):
    o_ref[...] = x_ref[...].reshape(NUM_FACTOR, BLK // 128, 128)


def _detile(pt):
    return pl.pallas_call(
        _detile_body,
        grid=(NBLK,),
        in_specs=[pl.BlockSpec((NUM_FACTOR, BLK), lambda k: (0, k))],
        out_specs=pl.BlockSpec((NUM_FACTOR, BLK // 128, 128),
                               lambda k: (k, 0, 0)),
        out_shape=jax.ShapeDtypeStruct((NBLK * NUM_FACTOR, BLK // 128, 128),
                                       jnp.float32),
    )(pt)


def _mf_body(user_hbm, item_hbm, fp_hbm, fq_hbm, bu_hbm, bi_hbm, out_hbm,
             uidx_v, iidx_v, ids_p, ids_q, dp_v, dq_v,
             bu_v, bi_v, out_v,
             sem_bu, sem_bi, sem_p, sem_q):
    wid = lax.axis_index("s") * NC + lax.axis_index("c")
    base = wid * B_PER_W

    pltpu.sync_copy(user_hbm.at[pl.ds(base, B_PER_W)], uidx_v)
    pltpu.sync_copy(item_hbm.at[pl.ds(base, B_PER_W)], iidx_v)

    copies = [
        pltpu.async_copy(bu_hbm.at[uidx_v], bu_v, sem_bu),
        pltpu.async_copy(bi_hbm.at[iidx_v], bi_v, sem_bi),
    ]

    def build(g, _):
        for idxv, ids in ((uidx_v, ids_p), (iidx_v, ids_q)):
            u16 = idxv[pl.ds(g * L, L)]
            a0 = (u16 >> 14) * (NUM_FACTOR * BLK) + (u16 & (BLK - 1))
            for f in range(NUM_FACTOR):
                ids[pl.ds(f * B_PER_W + g * L, L)] = a0 + f * BLK
        return 0

    lax.fori_loop(0, B_PER_W // L, build, 0)

    copies.append(pltpu.async_copy(fp_hbm.at[ids_p], dp_v, sem_p))
    copies.append(pltpu.async_copy(fq_hbm.at[ids_q], dq_v, sem_q))
    for c in copies:
        c.wait()

    def dot(g, _):
        sl = pl.ds(g * L, L)
        acc = bu_v[sl] + bi_v[sl]
        for f in range(NUM_FACTOR):
            pv = dp_v[pl.ds(f * B_PER_W + g * L, L)]
            qv = dq_v[pl.ds(f * B_PER_W + g * L, L)]
            acc = acc + pv * qv
        out_v[sl] = acc
        return 0

    lax.fori_loop(0, B_PER_W // L, dot, 0)

    pltpu.sync_copy(out_v, out_hbm.at[pl.ds(base, B_PER_W)])


@jax.jit
def _mf(user, item, p, q, b_u, b_i):
    fp = _detile(jnp.transpose(p)).reshape(NBLK * NUM_FACTOR * (BLK // 128) * 128)
    fq = _detile(jnp.transpose(q)).reshape(NBLK * NUM_FACTOR * (BLK // 128) * 128)
    mesh = plsc.VectorSubcoreMesh(
        core_axis_name="c", subcore_axis_name="s",
        num_cores=NC, num_subcores=NS)
    return pl.kernel(
        _mf_body,
        out_type=jax.ShapeDtypeStruct((BATCH,), jnp.float32),
        mesh=mesh,
        compiler_params=pltpu.CompilerParams(
            needs_layout_passes=False, use_tc_tiling_on_sc=False),
        scratch_types=[
            pltpu.VMEM((B_PER_W,), jnp.int32),
            pltpu.VMEM((B_PER_W,), jnp.int32),
            pltpu.VMEM((N_GATHER,), jnp.int32),
            pltpu.VMEM((N_GATHER,), jnp.int32),
            pltpu.VMEM((N_GATHER,), jnp.float32),
            pltpu.VMEM((N_GATHER,), jnp.float32),
            pltpu.VMEM((B_PER_W,), jnp.float32),
            pltpu.VMEM((B_PER_W,), jnp.float32),
            pltpu.VMEM((B_PER_W,), jnp.float32),
        ] + [pltpu.SemaphoreType.DMA] * 4,
    )(user, item, fp, fq, b_u, b_i)


def kernel(user, item, p, q, b_u, b_i):
    return _mf(user, item, p, q, b_u, b_i)


# speedup vs baseline: 8.8844x; 1.0885x over previous
import jax
import jax.numpy as jnp
from jax import lax
from jax.experimental import pallas as pl
from jax.experimental.pallas import tpu as pltpu
from jax.experimental.pallas import tpu_sc as plsc

NUM_FACTOR = 32
BATCH = 16384
NUM_ROWS = 1000000
NC = 2
NS = 16
L = 16
NW = NC * NS
B_PER_W = BATCH // NW
N_GATHER = B_PER_W * NUM_FACTOR
BLK = 65536
NBLK = (NUM_ROWS + BLK - 1) // BLK


def _detile_body(x_ref, o_ref):
    o_ref[...] = x_ref[...].reshape(NUM_FACTOR, BLK // 128, 128)


def _detile(pt):
    return pl.pallas_call(
        _detile_body,
        grid=(NBLK,),
        in_specs=[pl.BlockSpec((NUM_FACTOR, BLK), lambda k: (0, k))],
        out_specs=pl.BlockSpec((NUM_FACTOR, BLK // 128, 128),
                               lambda k: (k, 0, 0)),
        out_shape=jax.ShapeDtypeStruct((NBLK * NUM_FACTOR, BLK // 128, 128),
                                       jnp.float32),
    )(pt)


def _mf_body(user_hbm, item_hbm, fp_hbm, fq_hbm, bu_hbm, bi_hbm, out_hbm,
             uidx_v, iidx_v, ids_p, ids_q, dp_v, dq_v,
             bu_v, bi_v, out_v,
             sem_bu, sem_bi, sem_p, sem_q):
    wid = lax.axis_index("s") * NC + lax.axis_index("c")
    base = wid * B_PER_W

    pltpu.sync_copy(user_hbm.at[pl.ds(base, B_PER_W)], uidx_v)
    pltpu.sync_copy(item_hbm.at[pl.ds(base, B_PER_W)], iidx_v)

    copies = [
        pltpu.async_copy(bu_hbm.at[uidx_v], bu_v, sem_bu),
        pltpu.async_copy(bi_hbm.at[iidx_v], bi_v, sem_bi),
    ]

    def build(g, _):
        for idxv, ids in ((uidx_v, ids_p), (iidx_v, ids_q)):
            u16 = idxv[pl.ds(g * L, L)]
            a0 = (u16 >> 16) * (NUM_FACTOR * BLK) + (u16 & (BLK - 1))
            for f in range(NUM_FACTOR):
                ids[pl.ds(f * B_PER_W + g * L, L)] = a0 + f * BLK
        return 0

    lax.fori_loop(0, B_PER_W // L, build, 0)

    copies.append(pltpu.async_copy(fp_hbm.at[ids_p], dp_v, sem_p))
    copies.append(pltpu.async_copy(fq_hbm.at[ids_q], dq_v, sem_q))
    for c in copies:
        c.wait()

    def dot(g, _):
        sl = pl.ds(g * L, L)
        acc = bu_v[sl] + bi_v[sl]
        for f in range(NUM_FACTOR):
            pv = dp_v[pl.ds(f * B_PER_W + g * L, L)]
            qv = dq_v[pl.ds(f * B_PER_W + g * L, L)]
            acc = acc + pv * qv
        out_v[sl] = acc
        return 0

    lax.fori_loop(0, B_PER_W // L, dot, 0)

    pltpu.sync_copy(out_v, out_hbm.at[pl.ds(base, B_PER_W)])


@jax.jit
def _mf(user, item, p, q, b_u, b_i):
    fp = _detile(jnp.transpose(p)).reshape(NBLK * NUM_FACTOR * (BLK // 128) * 128)
    fq = _detile(jnp.transpose(q)).reshape(NBLK * NUM_FACTOR * (BLK // 128) * 128)
    mesh = plsc.VectorSubcoreMesh(
        core_axis_name="c", subcore_axis_name="s",
        num_cores=NC, num_subcores=NS)
    return pl.kernel(
        _mf_body,
        out_type=jax.ShapeDtypeStruct((BATCH,), jnp.float32),
        mesh=mesh,
        compiler_params=pltpu.CompilerParams(
            needs_layout_passes=False, use_tc_tiling_on_sc=False),
        scratch_types=[
            pltpu.VMEM((B_PER_W,), jnp.int32),
            pltpu.VMEM((B_PER_W,), jnp.int32),
            pltpu.VMEM((N_GATHER,), jnp.int32),
            pltpu.VMEM((N_GATHER,), jnp.int32),
            pltpu.VMEM((N_GATHER,), jnp.float32),
            pltpu.VMEM((N_GATHER,), jnp.float32),
            pltpu.VMEM((B_PER_W,), jnp.float32),
            pltpu.VMEM((B_PER_W,), jnp.float32),
            pltpu.VMEM((B_PER_W,), jnp.float32),
        ] + [pltpu.SemaphoreType.DMA] * 4,
    )(user, item, fp, fq, b_u, b_i)


def kernel(user, item, p, q, b_u, b_i):
    return _mf(user, item, p, q, b_u, b_i)
